# baseline (speedup 1.0000x reference)
import jax
import jax.numpy as jnp
from jax.experimental import pallas as pl
from jax.experimental.pallas import tpu as pltpu

_DELTA = 0.001
_BK = 5000


def _idw_block(q_ref, keys_ref, out_ref, qsqd_ref):
    @pl.when(pl.program_id(0) == 0)
    def _():
        q0 = q_ref[...]
        qsqd_ref[...] = jnp.sum(q0 * q0, axis=1, keepdims=True).T + _DELTA

    k = keys_ref[...]
    k_sq = jnp.sum(k * k, axis=1, keepdims=True)
    cross = jax.lax.dot_general(
        (k * -2.0).astype(jnp.bfloat16),
        q_ref[...].astype(jnp.bfloat16),
        (((1,), (1,)), ((), ())),
        preferred_element_type=jnp.float32,
    )
    sq = (cross + k_sq) + qsqd_ref[...]
    out_ref[...] = pl.reciprocal(jnp.maximum(sq, _DELTA), approx=True)


def kernel(key, keys):
    q_n, d = key.shape
    k_n = keys.shape[0]
    grid = (pl.cdiv(k_n, _BK),)
    out_t = pl.pallas_call(
        _idw_block,
        grid=grid,
        in_specs=[
            pl.BlockSpec((q_n, d), lambda i: (0, 0)),
            pl.BlockSpec((_BK, d), lambda i: (i, 0)),
        ],
        out_specs=pl.BlockSpec((_BK, q_n), lambda i: (i, 0)),
        out_shape=jax.ShapeDtypeStruct((k_n, q_n), jnp.float32),
        scratch_shapes=[pltpu.VMEM((1, q_n), jnp.float32)],
    )(key, keys)
    return jnp.swapaxes(out_t, 0, 1)
